# Initial kernel scaffold; baseline (speedup 1.0000x reference)
import jax
import jax.numpy as jnp
from jax.experimental import pallas as pl


def kernel(z0, prev_mean, dz, Wa, wv, eps):
    raise NotImplementedError("write your pallas kernel here")


# speedup vs baseline: 1.1816x; 1.1816x over previous
import functools

import jax
import jax.numpy as jnp
from jax import lax
from jax.experimental import pallas as pl

T = 16
A = 32
L = 64
N = 32768
K = 1024
ITERS = 2
MIN_STD = 0.05
MAX_STD = 2.0
TEMP = 0.5
RHO = 0.99

D = T * A
B1 = 2048
ROWS = N // D

_INT_MIN = -(2 ** 31)
_POS_HI = 0x7F800000
_NEG_LO = -2139095041


def _sortable_key(s):
    i = lax.bitcast_convert_type(s, jnp.int32)
    return jnp.where(i >= 0, i, jnp.bitwise_not(i ^ jnp.int32(_INT_MIN)))


def _k1_body(eps_ref, s1_ref, s2_ref, z0_ref, dz_ref, wa_ref, wv_ref,
             sc_ref, mean_ref, std_ref):
    mean = jnp.sum(s1_ref[...], axis=0, keepdims=True)
    es2 = jnp.sum(s2_ref[...], axis=0, keepdims=True)
    var = es2 - mean * mean
    std = jnp.clip(jnp.sqrt(jnp.clip(var, 0.0, None)), MIN_STD, MAX_STD)

    @pl.when(pl.program_id(0) == 0)
    def _():
        mean_ref[...] = mean
        std_ref[...] = std

    a = jnp.clip(mean + std * eps_ref[...], -1.0, 1.0)
    z = jnp.broadcast_to(z0_ref[...], (B1, L))
    dz = dz_ref[...]
    val = jnp.zeros((B1, 1), jnp.float32)
    disc = 1.0
    for t in range(T):
        at = a[:, t * A:(t + 1) * A]
        z = jnp.tanh(z * dz + jnp.dot(at, wa_ref[...],
                                      preferred_element_type=jnp.float32))
        val = val + disc * jnp.dot(z, wv_ref[...],
                                   preferred_element_type=jnp.float32)
        disc = disc * RHO
    sc_ref[...] = val


def _scores(eps2d, s1p, s2p, z0r, dzr, wa, wvr):
    grid = N // B1
    return pl.pallas_call(
        _k1_body,
        grid=(grid,),
        in_specs=[
            pl.BlockSpec((B1, D), lambda i: (i, 0)),
            pl.BlockSpec(s1p.shape, lambda i: (0, 0)),
            pl.BlockSpec(s2p.shape, lambda i: (0, 0)),
            pl.BlockSpec((1, L), lambda i: (0, 0)),
            pl.BlockSpec((1, L), lambda i: (0, 0)),
            pl.BlockSpec((A, L), lambda i: (0, 0)),
            pl.BlockSpec((L, 1), lambda i: (0, 0)),
        ],
        out_specs=[
            pl.BlockSpec((B1, 1), lambda i: (i, 0)),
            pl.BlockSpec((1, D), lambda i: (0, 0)),
            pl.BlockSpec((1, D), lambda i: (0, 0)),
        ],
        out_shape=[
            jax.ShapeDtypeStruct((N, 1), jnp.float32),
            jax.ShapeDtypeStruct((1, D), jnp.float32),
            jax.ShapeDtypeStruct((1, D), jnp.float32),
        ],
    )(eps2d, s1p, s2p, z0r, dzr, wa, wvr)


def _k2_body(s_ref, w_ref):
    s = s_ref[...]
    key = _sortable_key(s)
    kf = jnp.float32(K)

    def cnt_ge(t):
        return jnp.sum((key >= t).astype(jnp.float32))

    cnt0 = cnt_ge(jnp.int32(0))
    lo0 = jnp.where(cnt0 >= kf, jnp.int32(0), jnp.int32(_NEG_LO))
    hi0 = jnp.where(cnt0 >= kf, jnp.int32(_POS_HI), jnp.int32(-1))

    def body(_, lh):
        lo, hi = lh
        mid = lo + ((hi - lo + 1) >> 1)
        p = cnt_ge(mid) >= kf
        return (jnp.where(p, mid, lo), jnp.where(p, hi, mid - 1))

    theta, _ = lax.fori_loop(0, 31, body, (lo0, hi0))

    gt = key > theta
    eq = key == theta
    cgt = jnp.sum(gt.astype(jnp.float32))
    needed = kf - cgt
    eqf = eq.astype(jnp.float32)
    li = lax.broadcasted_iota(jnp.int32, (D, D), 0)
    pi = lax.broadcasted_iota(jnp.int32, (D, D), 1)
    upper = (li <= pi).astype(jnp.float32)
    prefix = jnp.dot(eqf, upper, preferred_element_type=jnp.float32)
    tot = prefix[:, D - 1:D]
    ri = lax.broadcasted_iota(jnp.int32, (ROWS, ROWS), 0)
    ci = lax.broadcasted_iota(jnp.int32, (ROWS, ROWS), 1)
    lstrict = (ci < ri).astype(jnp.float32)
    rowoff = jnp.dot(lstrict, tot, preferred_element_type=jnp.float32)
    grank = prefix + rowoff
    sel = gt | (eq & (grank <= needed))

    m = jnp.max(s)
    inv_t = 1.0 / TEMP
    p = jnp.where(sel, jnp.exp(s * inv_t - m * inv_t), 0.0)
    w_ref[...] = p / jnp.sum(p)


def _weights(s2d):
    return pl.pallas_call(
        _k2_body,
        out_shape=jax.ShapeDtypeStruct((ROWS, D), jnp.float32),
    )(s2d)


def _k3_body(eps_ref, w_ref, mean_ref, std_ref, s1_ref, s2_ref):
    a = jnp.clip(mean_ref[...] + std_ref[...] * eps_ref[...], -1.0, 1.0)
    w = w_ref[...]
    dn = (((0,), (0,)), ((), ()))
    wa = lax.dot_general(w, a, dn, preferred_element_type=jnp.float32)
    waa = lax.dot_general(w, a * a, dn, preferred_element_type=jnp.float32)

    @pl.when(pl.program_id(0) == 0)
    def _():
        s1_ref[...] = wa
        s2_ref[...] = waa

    @pl.when(pl.program_id(0) != 0)
    def _():
        s1_ref[...] += wa
        s2_ref[...] += waa


def _elite_update(eps2d, wcol, meanr, stdr):
    grid = N // B1
    return pl.pallas_call(
        _k3_body,
        grid=(grid,),
        in_specs=[
            pl.BlockSpec((B1, D), lambda i: (i, 0)),
            pl.BlockSpec((B1, 1), lambda i: (i, 0)),
            pl.BlockSpec((1, D), lambda i: (0, 0)),
            pl.BlockSpec((1, D), lambda i: (0, 0)),
        ],
        out_specs=[
            pl.BlockSpec((1, D), lambda i: (0, 0)),
            pl.BlockSpec((1, D), lambda i: (0, 0)),
        ],
        out_shape=[
            jax.ShapeDtypeStruct((1, D), jnp.float32),
            jax.ShapeDtypeStruct((1, D), jnp.float32),
        ],
    )(eps2d, wcol, meanr, stdr)


@jax.jit
def kernel(z0, prev_mean, dz, Wa, wv, eps):
    z0r = z0.reshape(1, L)
    dzr = dz.reshape(1, L)
    wvr = wv.reshape(L, 1)
    eps2d = eps.reshape(ITERS, N, D)

    shifted = jnp.zeros_like(prev_mean).at[:-1].set(prev_mean[1:])
    m0 = shifted.reshape(1, D)
    s1p = m0
    s2p = MAX_STD * MAX_STD + m0 * m0

    for it in range(ITERS):
        e = eps2d[it]
        scores, meanr, stdr = _scores(e, s1p, s2p, z0r, dzr, Wa, wvr)
        w2d = _weights(scores.reshape(ROWS, D))
        s1p, s2p = _elite_update(e, w2d.reshape(N, 1), meanr, stdr)

    mean_final = jnp.sum(s1p, axis=0).reshape(T, A)
    return mean_final
